# baseline (speedup 1.0000x reference)
import jax
import jax.numpy as jnp
from jax import lax
from jax.experimental import pallas as pl

S = 200
GN = 4096
D = 128
N_COMMANDS = 7
GROUP_VOCAB = 52
VOCAB_PAD = 64
G_BLK = 4096


def _body(cmd_ref, grp_ref, args_ref, w1_ref, w2_ref, b_ref, pos_ref, out_ref):
    c = cmd_ref[0]
    g = grp_ref[0]
    iota = lax.broadcasted_iota(jnp.int32, (VOCAB_PAD, 1), 0)
    oh_t = (iota == c).astype(jnp.float32) + (iota == g + N_COMMANDS).astype(jnp.float32)
    acc = lax.dot_general(
        oh_t, w1_ref[...], (((0,), (0,)), ((), ())),
        preferred_element_type=jnp.float32,
    )
    acc = acc + jnp.dot(args_ref[0], w2_ref[...],
                        preferred_element_type=jnp.float32)
    pb = pos_ref[0] + b_ref[...]
    out_ref[0] = acc + pb


def kernel(commands, args, groups, command_embed, W_fcn, b_fcn, group_embed, pos_embed):
    w1 = jnp.concatenate(
        [command_embed, group_embed,
         jnp.zeros((VOCAB_PAD - N_COMMANDS - GROUP_VOCAB, D), jnp.float32)], axis=0)
    w2 = W_fcn.T
    b2 = b_fcn.reshape(1, D)
    cmd3 = commands.reshape(S, 1, GN).astype(jnp.int32)
    grp3 = groups.reshape(S, 1, GN).astype(jnp.int32)
    pos3 = pos_embed.reshape(-1, 1, D)

    grid = (S, GN // G_BLK)
    out = pl.pallas_call(
        _body,
        grid=grid,
        in_specs=[
            pl.BlockSpec((1, 1, G_BLK), lambda s, j: (s, 0, j)),
            pl.BlockSpec((1, 1, G_BLK), lambda s, j: (s, 0, j)),
            pl.BlockSpec((1, G_BLK, args.shape[-1]), lambda s, j: (s, j, 0)),
            pl.BlockSpec((VOCAB_PAD, D), lambda s, j: (0, 0)),
            pl.BlockSpec((W_fcn.shape[1], D), lambda s, j: (0, 0)),
            pl.BlockSpec((1, D), lambda s, j: (0, 0)),
            pl.BlockSpec((1, 1, D), lambda s, j: (s, 0, 0)),
        ],
        out_specs=pl.BlockSpec((1, G_BLK, D), lambda s, j: (s, j, 0)),
        out_shape=jax.ShapeDtypeStruct((S, GN, D), jnp.float32),
    )(cmd3, grp3, args, w1, w2, b2, pos3)
    return out


# speedup vs baseline: 1.1150x; 1.1150x over previous
import jax
import jax.numpy as jnp
from jax import lax
from jax.experimental import pallas as pl

S = 200
GN = 4096
D = 128
N_COMMANDS = 7
GROUP_VOCAB = 52
VOCAB_PAD = 64
ROWS = 2


def _body(cmd_ref, grp_ref, args_ref, w1_ref, w2_ref, b_ref, pos_ref, out_ref):
    for r in range(ROWS):
        c = cmd_ref[r]
        g = grp_ref[r]
        iota = lax.broadcasted_iota(jnp.int32, (VOCAB_PAD, 1), 0)
        oh_t = (iota == c).astype(jnp.float32) + (iota == g + N_COMMANDS).astype(jnp.float32)
        acc = lax.dot_general(
            oh_t, w1_ref[...], (((0,), (0,)), ((), ())),
            preferred_element_type=jnp.float32,
        )
        acc = acc + jnp.dot(args_ref[r], w2_ref[...],
                            preferred_element_type=jnp.float32)
        pb = pos_ref[r] + b_ref[...]
        out_ref[r] = acc + pb


def kernel(commands, args, groups, command_embed, W_fcn, b_fcn, group_embed, pos_embed):
    w1 = jnp.concatenate(
        [command_embed, group_embed,
         jnp.zeros((VOCAB_PAD - N_COMMANDS - GROUP_VOCAB, D), jnp.float32)], axis=0)
    w2 = W_fcn.T
    b2 = b_fcn.reshape(1, D)
    cmd3 = commands.reshape(S, 1, GN).astype(jnp.int32)
    grp3 = groups.reshape(S, 1, GN).astype(jnp.int32)
    pos3 = pos_embed.reshape(-1, 1, D)

    grid = (S // ROWS,)
    out = pl.pallas_call(
        _body,
        grid=grid,
        in_specs=[
            pl.BlockSpec((ROWS, 1, GN), lambda s: (s, 0, 0)),
            pl.BlockSpec((ROWS, 1, GN), lambda s: (s, 0, 0)),
            pl.BlockSpec((ROWS, GN, args.shape[-1]), lambda s: (s, 0, 0)),
            pl.BlockSpec((VOCAB_PAD, D), lambda s: (0, 0)),
            pl.BlockSpec((W_fcn.shape[1], D), lambda s: (0, 0)),
            pl.BlockSpec((1, D), lambda s: (0, 0)),
            pl.BlockSpec((ROWS, 1, D), lambda s: (s, 0, 0)),
        ],
        out_specs=pl.BlockSpec((ROWS, GN, D), lambda s: (s, 0, 0)),
        out_shape=jax.ShapeDtypeStruct((S, GN, D), jnp.float32),
    )(cmd3, grp3, args, w1, w2, b2, pos3)
    return out


# speedup vs baseline: 1.1375x; 1.0202x over previous
import jax
import jax.numpy as jnp
from jax import lax
from jax.experimental import pallas as pl

S = 200
GN = 4096
D = 128
N_COMMANDS = 7
GROUP_VOCAB = 52
VOCAB_PAD = 64
ROWS = 4


def _body(cmd_ref, grp_ref, args_ref, w1_ref, w2_ref, b_ref, pos_ref, out_ref):
    for r in range(ROWS):
        c = cmd_ref[r]
        g = grp_ref[r]
        iota = lax.broadcasted_iota(jnp.int32, (VOCAB_PAD, 1), 0)
        oh_t = (iota == c).astype(jnp.float32) + (iota == g + N_COMMANDS).astype(jnp.float32)
        acc = lax.dot_general(
            oh_t, w1_ref[...], (((0,), (0,)), ((), ())),
            preferred_element_type=jnp.float32,
        )
        acc = acc + jnp.dot(args_ref[r], w2_ref[...],
                            preferred_element_type=jnp.float32)
        pb = pos_ref[r] + b_ref[...]
        out_ref[r] = acc + pb


def kernel(commands, args, groups, command_embed, W_fcn, b_fcn, group_embed, pos_embed):
    w1 = jnp.concatenate(
        [command_embed, group_embed,
         jnp.zeros((VOCAB_PAD - N_COMMANDS - GROUP_VOCAB, D), jnp.float32)], axis=0)
    w2 = W_fcn.T
    b2 = b_fcn.reshape(1, D)
    cmd3 = commands.reshape(S, 1, GN).astype(jnp.int32)
    grp3 = groups.reshape(S, 1, GN).astype(jnp.int32)
    pos3 = pos_embed.reshape(-1, 1, D)

    grid = (S // ROWS,)
    out = pl.pallas_call(
        _body,
        grid=grid,
        in_specs=[
            pl.BlockSpec((ROWS, 1, GN), lambda s: (s, 0, 0)),
            pl.BlockSpec((ROWS, 1, GN), lambda s: (s, 0, 0)),
            pl.BlockSpec((ROWS, GN, args.shape[-1]), lambda s: (s, 0, 0)),
            pl.BlockSpec((VOCAB_PAD, D), lambda s: (0, 0)),
            pl.BlockSpec((W_fcn.shape[1], D), lambda s: (0, 0)),
            pl.BlockSpec((1, D), lambda s: (0, 0)),
            pl.BlockSpec((ROWS, 1, D), lambda s: (s, 0, 0)),
        ],
        out_specs=pl.BlockSpec((ROWS, GN, D), lambda s: (s, 0, 0)),
        out_shape=jax.ShapeDtypeStruct((S, GN, D), jnp.float32),
    )(cmd3, grp3, args, w1, w2, b2, pos3)
    return out


# speedup vs baseline: 1.1401x; 1.0023x over previous
import jax
import jax.numpy as jnp
from jax import lax
from jax.experimental import pallas as pl

S = 200
GN = 4096
D = 128
N_COMMANDS = 7
GROUP_VOCAB = 52
VOCAB_PAD = 64
ROWS = 5


def _body(cmd_ref, grp_ref, args_ref, w1_ref, w2_ref, b_ref, pos_ref, out_ref):
    for r in range(ROWS):
        c = cmd_ref[r]
        g = grp_ref[r]
        iota = lax.broadcasted_iota(jnp.int32, (VOCAB_PAD, 1), 0)
        oh_t = (iota == c).astype(jnp.float32) + (iota == g + N_COMMANDS).astype(jnp.float32)
        acc = lax.dot_general(
            oh_t, w1_ref[...], (((0,), (0,)), ((), ())),
            preferred_element_type=jnp.float32,
        )
        acc = acc + jnp.dot(args_ref[r], w2_ref[...],
                            preferred_element_type=jnp.float32)
        pb = pos_ref[r] + b_ref[...]
        out_ref[r] = acc + pb


def kernel(commands, args, groups, command_embed, W_fcn, b_fcn, group_embed, pos_embed):
    w1 = jnp.concatenate(
        [command_embed, group_embed,
         jnp.zeros((VOCAB_PAD - N_COMMANDS - GROUP_VOCAB, D), jnp.float32)], axis=0)
    w2 = W_fcn.T
    b2 = b_fcn.reshape(1, D)
    cmd3 = commands.reshape(S, 1, GN).astype(jnp.int32)
    grp3 = groups.reshape(S, 1, GN).astype(jnp.int32)
    pos3 = pos_embed.reshape(-1, 1, D)

    grid = (S // ROWS,)
    out = pl.pallas_call(
        _body,
        grid=grid,
        in_specs=[
            pl.BlockSpec((ROWS, 1, GN), lambda s: (s, 0, 0)),
            pl.BlockSpec((ROWS, 1, GN), lambda s: (s, 0, 0)),
            pl.BlockSpec((ROWS, GN, args.shape[-1]), lambda s: (s, 0, 0)),
            pl.BlockSpec((VOCAB_PAD, D), lambda s: (0, 0)),
            pl.BlockSpec((W_fcn.shape[1], D), lambda s: (0, 0)),
            pl.BlockSpec((1, D), lambda s: (0, 0)),
            pl.BlockSpec((ROWS, 1, D), lambda s: (s, 0, 0)),
        ],
        out_specs=pl.BlockSpec((ROWS, GN, D), lambda s: (s, 0, 0)),
        out_shape=jax.ShapeDtypeStruct((S, GN, D), jnp.float32),
    )(cmd3, grp3, args, w1, w2, b2, pos3)
    return out
